# baseline (speedup 1.0000x reference)
import functools

import jax
import jax.numpy as jnp
from jax import lax
from jax.experimental import pallas as pl
from jax.experimental.pallas import tpu as pltpu
from jax.experimental.pallas import tpu_sc as plsc

NQ = 4096
NK = 10475
LANES = 16
NKPAD = 10496
NCHUNK = NKPAD // LANES
NC = 2
NS = 16
NW = NC * NS

SCQ = 1024
TCQ = NQ - SCQ
QPW = SCQ // NW
TC_BLK = 128

_BIG = 3.0e38
_BIGI = 2**30


@functools.partial(
    pl.kernel,
    out_type=jax.ShapeDtypeStruct((SCQ,), jnp.int32),
    mesh=plsc.VectorSubcoreMesh(core_axis_name="c", subcore_axis_name="s"),
    scratch_types=[
        pltpu.VMEM((NKPAD,), jnp.float32),
        pltpu.VMEM((NKPAD,), jnp.float32),
        pltpu.VMEM((NKPAD,), jnp.float32),
        pltpu.VMEM((QPW,), jnp.float32),
        pltpu.VMEM((QPW,), jnp.float32),
        pltpu.VMEM((QPW,), jnp.float32),
        pltpu.VMEM((QPW,), jnp.int32),
    ],
    compiler_params=pltpu.CompilerParams(needs_layout_passes=False),
)
def _match_sc(qx_hbm, qy_hbm, qz_hbm, kx_hbm, ky_hbm, kz_hbm, out_hbm,
              kxv, kyv, kzv, qxv, qyv, qzv, outv):
    wid = lax.axis_index("s") * NC + lax.axis_index("c")
    base = wid * QPW

    pltpu.sync_copy(kx_hbm, kxv)
    pltpu.sync_copy(ky_hbm, kyv)
    pltpu.sync_copy(kz_hbm, kzv)
    pltpu.sync_copy(qx_hbm.at[pl.ds(base, QPW)], qxv)
    pltpu.sync_copy(qy_hbm.at[pl.ds(base, QPW)], qyv)
    pltpu.sync_copy(qz_hbm.at[pl.ds(base, QPW)], qzv)

    lane_iota = lax.iota(jnp.int32, LANES)

    def group_body(g, carry0):
        acc = jnp.zeros((LANES,), jnp.int32)
        qgx = qxv[pl.ds(g * LANES, LANES)]
        qgy = qyv[pl.ds(g * LANES, LANES)]
        qgz = qzv[pl.ds(g * LANES, LANES)]
        for l in range(LANES):
            qxs = jnp.full((LANES,), qgx[l], jnp.float32)
            qys = jnp.full((LANES,), qgy[l], jnp.float32)
            qzs = jnp.full((LANES,), qgz[l], jnp.float32)

            def key_body(c, carry, qxs=qxs, qys=qys, qzs=qzs):
                minv, mini, idxv = carry
                off = c * LANES
                dx = kxv[pl.ds(off, LANES)] - qxs
                dy = kyv[pl.ds(off, LANES)] - qys
                dz = kzv[pl.ds(off, LANES)] - qzs
                d = dx * dx + dy * dy + dz * dz
                pred = d < minv
                mini = jnp.where(pred, idxv, mini)
                minv = jnp.minimum(d, minv)
                return minv, mini, idxv + LANES

            init = (jnp.full((LANES,), _BIG, jnp.float32),
                    jnp.zeros((LANES,), jnp.int32),
                    lane_iota)
            minv, mini, _ = lax.fori_loop(0, NCHUNK, key_body, init, unroll=8)

            vbest = -plsc.cummax(-minv)[LANES - 1]
            cand = jnp.where(minv == vbest, mini, jnp.int32(_BIGI))
            ibest = -plsc.cummax(-cand)[LANES - 1]
            acc = jnp.where(lane_iota == l, ibest, acc)
        outv[pl.ds(g * LANES, LANES)] = acc
        return carry0

    lax.fori_loop(0, QPW // LANES, group_body, 0)
    pltpu.sync_copy(outv, out_hbm.at[pl.ds(base, QPW)])


TC_KCH = 128


def _tc_body(qx_ref, qy_ref, qz_ref, kx_ref, ky_ref, kz_ref, out_ref):
    qb = pl.multiple_of(pl.program_id(0) * TC_BLK, TC_BLK)
    qx = qx_ref[pl.ds(qb, TC_BLK)][:, None]
    qy = qy_ref[pl.ds(qb, TC_BLK)][:, None]
    qz = qz_ref[pl.ds(qb, TC_BLK)][:, None]
    lane = lax.broadcasted_iota(jnp.int32, (TC_BLK, TC_KCH), 1)

    def body(c, carry):
        minv, minc = carry
        off = c * TC_KCH
        dx = qx - kx_ref[pl.ds(off, TC_KCH)][None, :]
        dy = qy - ky_ref[pl.ds(off, TC_KCH)][None, :]
        dz = qz - kz_ref[pl.ds(off, TC_KCH)][None, :]
        d2 = dx * dx + dy * dy + dz * dz
        pred = d2 < minv
        minc = jnp.where(pred, c, minc)
        minv = jnp.minimum(d2, minv)
        return minv, minc

    init = (jnp.full((TC_BLK, TC_KCH), _BIG, jnp.float32),
            jnp.zeros((TC_BLK, TC_KCH), jnp.int32))
    minv, minc = lax.fori_loop(0, NKPAD // TC_KCH, body, init, unroll=2)
    mini = minc * TC_KCH + lane
    vbest = jnp.min(minv, axis=1, keepdims=True)
    cand = jnp.where(minv == vbest, mini, jnp.int32(_BIGI))
    out_ref[pl.ds(qb, TC_BLK)] = jnp.min(cand, axis=1).astype(jnp.int32)


_match_tc = pl.pallas_call(
    _tc_body,
    grid=(TCQ // TC_BLK,),
    in_specs=[
        pl.BlockSpec((TCQ,), lambda i: (0,)),
        pl.BlockSpec((TCQ,), lambda i: (0,)),
        pl.BlockSpec((TCQ,), lambda i: (0,)),
        pl.BlockSpec((NKPAD,), lambda i: (0,)),
        pl.BlockSpec((NKPAD,), lambda i: (0,)),
        pl.BlockSpec((NKPAD,), lambda i: (0,)),
    ],
    out_specs=pl.BlockSpec((TCQ,), lambda i: (0,)),
    out_shape=jax.ShapeDtypeStruct((TCQ,), jnp.int32),
)


def kernel(mhr_vertices, smplx_vertices):
    q = mhr_vertices.astype(jnp.float32)
    s = smplx_vertices.astype(jnp.float32)
    pad = NKPAD - NK
    kx = jnp.pad(s[:, 0], (0, pad), constant_values=1.0e9)
    ky = jnp.pad(s[:, 1], (0, pad), constant_values=1.0e9)
    kz = jnp.pad(s[:, 2], (0, pad), constant_values=1.0e9)
    out_sc = _match_sc(q[:SCQ, 0], q[:SCQ, 1], q[:SCQ, 2], kx, ky, kz)
    out_tc = _match_tc(q[SCQ:, 0], q[SCQ:, 1], q[SCQ:, 2], kx, ky, kz)
    return jnp.concatenate([out_sc, out_tc])


# speedup vs baseline: 1.0645x; 1.0645x over previous
import functools

import jax
import jax.numpy as jnp
from jax import lax
from jax.experimental import pallas as pl
from jax.experimental.pallas import tpu as pltpu
from jax.experimental.pallas import tpu_sc as plsc

NQ = 4096
NK = 10475
LANES = 16
NKPAD = 10496
NCHUNK = NKPAD // LANES
NC = 2
NS = 16
NW = NC * NS

SCQ = 1024
TCQ = NQ - SCQ
QPW = SCQ // NW
TC_BLK = 128

_BIG = 3.0e38
_BIGI = 2**30


@functools.partial(
    pl.kernel,
    out_type=jax.ShapeDtypeStruct((SCQ,), jnp.int32),
    mesh=plsc.VectorSubcoreMesh(core_axis_name="c", subcore_axis_name="s"),
    scratch_types=[
        pltpu.VMEM((NKPAD,), jnp.float32),
        pltpu.VMEM((NKPAD,), jnp.float32),
        pltpu.VMEM((NKPAD,), jnp.float32),
        pltpu.VMEM((QPW,), jnp.float32),
        pltpu.VMEM((QPW,), jnp.float32),
        pltpu.VMEM((QPW,), jnp.float32),
        pltpu.VMEM((QPW,), jnp.int32),
    ],
    compiler_params=pltpu.CompilerParams(needs_layout_passes=False),
)
def _match_sc(qx_hbm, qy_hbm, qz_hbm, kx_hbm, ky_hbm, kz_hbm, out_hbm,
              kxv, kyv, kzv, qxv, qyv, qzv, outv):
    wid = lax.axis_index("s") * NC + lax.axis_index("c")
    base = wid * QPW

    pltpu.sync_copy(kx_hbm, kxv)
    pltpu.sync_copy(ky_hbm, kyv)
    pltpu.sync_copy(kz_hbm, kzv)
    pltpu.sync_copy(qx_hbm.at[pl.ds(base, QPW)], qxv)
    pltpu.sync_copy(qy_hbm.at[pl.ds(base, QPW)], qyv)
    pltpu.sync_copy(qz_hbm.at[pl.ds(base, QPW)], qzv)

    lane_iota = lax.iota(jnp.int32, LANES)

    def group_body(g, carry0):
        acc = jnp.zeros((LANES,), jnp.int32)
        qgx = qxv[pl.ds(g * LANES, LANES)]
        qgy = qyv[pl.ds(g * LANES, LANES)]
        qgz = qzv[pl.ds(g * LANES, LANES)]
        for l in range(LANES):
            qxs = jnp.full((LANES,), qgx[l], jnp.float32)
            qys = jnp.full((LANES,), qgy[l], jnp.float32)
            qzs = jnp.full((LANES,), qgz[l], jnp.float32)

            def key_body(c, carry, qxs=qxs, qys=qys, qzs=qzs):
                minv, mini, idxv = carry
                off = c * LANES
                dx = kxv[pl.ds(off, LANES)] - qxs
                dy = kyv[pl.ds(off, LANES)] - qys
                dz = kzv[pl.ds(off, LANES)] - qzs
                d = dx * dx + dy * dy + dz * dz
                pred = d < minv
                mini = jnp.where(pred, idxv, mini)
                minv = jnp.minimum(d, minv)
                return minv, mini, idxv + LANES

            init = (jnp.full((LANES,), _BIG, jnp.float32),
                    jnp.zeros((LANES,), jnp.int32),
                    lane_iota)
            minv, mini, _ = lax.fori_loop(0, NCHUNK, key_body, init, unroll=8)

            vbest = -plsc.cummax(-minv)[LANES - 1]
            cand = jnp.where(minv == vbest, mini, jnp.int32(_BIGI))
            ibest = -plsc.cummax(-cand)[LANES - 1]
            acc = jnp.where(lane_iota == l, ibest, acc)
        outv[pl.ds(g * LANES, LANES)] = acc
        return carry0

    lax.fori_loop(0, QPW // LANES, group_body, 0)
    pltpu.sync_copy(outv, out_hbm.at[pl.ds(base, QPW)])


TC_KCH = 128


TC_SUB = 64


def _tc_body(qx_ref, qy_ref, qz_ref, kx_ref, ky_ref, kz_ref, out_ref):
    qb = pl.multiple_of(pl.program_id(0) * TC_BLK, TC_BLK)
    qxf = qx_ref[pl.ds(qb, TC_BLK)]
    qyf = qy_ref[pl.ds(qb, TC_BLK)]
    qzf = qz_ref[pl.ds(qb, TC_BLK)]
    lane = lax.broadcasted_iota(jnp.int32, (TC_SUB, TC_KCH), 1)

    results = []
    for h in range(TC_BLK // TC_SUB):
        qx = lax.slice(qxf, (h * TC_SUB,), ((h + 1) * TC_SUB,))[:, None]
        qy = lax.slice(qyf, (h * TC_SUB,), ((h + 1) * TC_SUB,))[:, None]
        qz = lax.slice(qzf, (h * TC_SUB,), ((h + 1) * TC_SUB,))[:, None]

        def body(c, carry, qx=qx, qy=qy, qz=qz):
            minv, minc = carry
            off = c * TC_KCH
            dx = qx - kx_ref[pl.ds(off, TC_KCH)][None, :]
            dy = qy - ky_ref[pl.ds(off, TC_KCH)][None, :]
            dz = qz - kz_ref[pl.ds(off, TC_KCH)][None, :]
            d2 = dx * dx + dy * dy + dz * dz
            pred = d2 < minv
            minc = jnp.where(pred, c, minc)
            minv = jnp.minimum(d2, minv)
            return minv, minc

        init = (jnp.full((TC_SUB, TC_KCH), _BIG, jnp.float32),
                jnp.zeros((TC_SUB, TC_KCH), jnp.int32))
        minv, minc = lax.fori_loop(0, NKPAD // TC_KCH, body, init, unroll=4)
        mini = minc * TC_KCH + lane
        vbest = jnp.min(minv, axis=1, keepdims=True)
        cand = jnp.where(minv == vbest, mini, jnp.int32(_BIGI))
        results.append(jnp.min(cand, axis=1).astype(jnp.int32))
    out_ref[pl.ds(qb, TC_BLK)] = jnp.concatenate(results)


_match_tc = pl.pallas_call(
    _tc_body,
    grid=(TCQ // TC_BLK,),
    in_specs=[
        pl.BlockSpec((TCQ,), lambda i: (0,)),
        pl.BlockSpec((TCQ,), lambda i: (0,)),
        pl.BlockSpec((TCQ,), lambda i: (0,)),
        pl.BlockSpec((NKPAD,), lambda i: (0,)),
        pl.BlockSpec((NKPAD,), lambda i: (0,)),
        pl.BlockSpec((NKPAD,), lambda i: (0,)),
    ],
    out_specs=pl.BlockSpec((TCQ,), lambda i: (0,)),
    out_shape=jax.ShapeDtypeStruct((TCQ,), jnp.int32),
)


def kernel(mhr_vertices, smplx_vertices):
    q = mhr_vertices.astype(jnp.float32)
    s = smplx_vertices.astype(jnp.float32)
    pad = NKPAD - NK
    kx = jnp.pad(s[:, 0], (0, pad), constant_values=1.0e9)
    ky = jnp.pad(s[:, 1], (0, pad), constant_values=1.0e9)
    kz = jnp.pad(s[:, 2], (0, pad), constant_values=1.0e9)
    out_sc = _match_sc(q[:SCQ, 0], q[:SCQ, 1], q[:SCQ, 2], kx, ky, kz)
    out_tc = _match_tc(q[SCQ:, 0], q[SCQ:, 1], q[SCQ:, 2], kx, ky, kz)
    return jnp.concatenate([out_sc, out_tc])


# speedup vs baseline: 1.1046x; 1.0376x over previous
import functools

import jax
import jax.numpy as jnp
from jax import lax
from jax.experimental import pallas as pl
from jax.experimental.pallas import tpu as pltpu
from jax.experimental.pallas import tpu_sc as plsc

NQ = 4096
NK = 10475
LANES = 16
NKPAD = 10496
NCHUNK = NKPAD // LANES
NC = 2
NS = 16
NW = NC * NS

SCQ = 1024
TCQ = NQ - SCQ
QPW = SCQ // NW
TC_BLK = 128

_BIG = 3.0e38
_BIGI = 2**30


@functools.partial(
    pl.kernel,
    out_type=jax.ShapeDtypeStruct((SCQ,), jnp.int32),
    mesh=plsc.VectorSubcoreMesh(core_axis_name="c", subcore_axis_name="s"),
    scratch_types=[
        pltpu.VMEM((NKPAD,), jnp.float32),
        pltpu.VMEM((NKPAD,), jnp.float32),
        pltpu.VMEM((NKPAD,), jnp.float32),
        pltpu.VMEM((QPW,), jnp.float32),
        pltpu.VMEM((QPW,), jnp.float32),
        pltpu.VMEM((QPW,), jnp.float32),
        pltpu.VMEM((QPW,), jnp.int32),
    ],
    compiler_params=pltpu.CompilerParams(needs_layout_passes=False),
)
def _match_sc(qx_hbm, qy_hbm, qz_hbm, kx_hbm, ky_hbm, kz_hbm, out_hbm,
              kxv, kyv, kzv, qxv, qyv, qzv, outv):
    wid = lax.axis_index("s") * NC + lax.axis_index("c")
    base = wid * QPW

    pltpu.sync_copy(kx_hbm, kxv)
    pltpu.sync_copy(ky_hbm, kyv)
    pltpu.sync_copy(kz_hbm, kzv)
    pltpu.sync_copy(qx_hbm.at[pl.ds(base, QPW)], qxv)
    pltpu.sync_copy(qy_hbm.at[pl.ds(base, QPW)], qyv)
    pltpu.sync_copy(qz_hbm.at[pl.ds(base, QPW)], qzv)

    lane_iota = lax.iota(jnp.int32, LANES)

    def group_body(g, carry0):
        acc = jnp.zeros((LANES,), jnp.int32)
        qgx = qxv[pl.ds(g * LANES, LANES)]
        qgy = qyv[pl.ds(g * LANES, LANES)]
        qgz = qzv[pl.ds(g * LANES, LANES)]
        for l in range(LANES):
            qxs = jnp.full((LANES,), qgx[l], jnp.float32)
            qys = jnp.full((LANES,), qgy[l], jnp.float32)
            qzs = jnp.full((LANES,), qgz[l], jnp.float32)

            def key_body(c, carry, qxs=qxs, qys=qys, qzs=qzs):
                minv, mini, idxv = carry
                off = c * LANES
                dx = kxv[pl.ds(off, LANES)] - qxs
                dy = kyv[pl.ds(off, LANES)] - qys
                dz = kzv[pl.ds(off, LANES)] - qzs
                d = dx * dx + dy * dy + dz * dz
                pred = d < minv
                mini = jnp.where(pred, idxv, mini)
                minv = jnp.minimum(d, minv)
                return minv, mini, idxv + LANES

            init = (jnp.full((LANES,), _BIG, jnp.float32),
                    jnp.zeros((LANES,), jnp.int32),
                    lane_iota)
            minv, mini, _ = lax.fori_loop(0, NCHUNK, key_body, init, unroll=8)

            vbest = -plsc.cummax(-minv)[LANES - 1]
            cand = jnp.where(minv == vbest, mini, jnp.int32(_BIGI))
            ibest = -plsc.cummax(-cand)[LANES - 1]
            acc = jnp.where(lane_iota == l, ibest, acc)
        outv[pl.ds(g * LANES, LANES)] = acc
        return carry0

    lax.fori_loop(0, QPW // LANES, group_body, 0)
    pltpu.sync_copy(outv, out_hbm.at[pl.ds(base, QPW)])


TC_KCH = 128


TC_SUB = 64


def _tc_body(qx_ref, qy_ref, qz_ref, kx_ref, ky_ref, kz_ref, out_ref):
    qb = pl.multiple_of(pl.program_id(0) * TC_BLK, TC_BLK)
    qxf = qx_ref[pl.ds(qb, TC_BLK)]
    qyf = qy_ref[pl.ds(qb, TC_BLK)]
    qzf = qz_ref[pl.ds(qb, TC_BLK)]
    lane = lax.broadcasted_iota(jnp.int32, (TC_SUB, TC_KCH), 1)

    results = []
    for h in range(TC_BLK // TC_SUB):
        qx = lax.slice(qxf, (h * TC_SUB,), ((h + 1) * TC_SUB,))[:, None]
        qy = lax.slice(qyf, (h * TC_SUB,), ((h + 1) * TC_SUB,))[:, None]
        qz = lax.slice(qzf, (h * TC_SUB,), ((h + 1) * TC_SUB,))[:, None]

        def body(c, carry, qx=qx, qy=qy, qz=qz):
            minv, minc = carry
            off = c * TC_KCH
            dx = qx - kx_ref[pl.ds(off, TC_KCH)][None, :]
            dy = qy - ky_ref[pl.ds(off, TC_KCH)][None, :]
            dz = qz - kz_ref[pl.ds(off, TC_KCH)][None, :]
            d2 = dx * dx + dy * dy + dz * dz
            pred = d2 < minv
            minc = jnp.where(pred, c, minc)
            minv = jnp.minimum(d2, minv)
            return minv, minc

        init = (jnp.full((TC_SUB, TC_KCH), _BIG, jnp.float32),
                jnp.zeros((TC_SUB, TC_KCH), jnp.int32))
        minv, minc = lax.fori_loop(0, NKPAD // TC_KCH, body, init, unroll=8)
        mini = minc * TC_KCH + lane
        vbest = jnp.min(minv, axis=1, keepdims=True)
        cand = jnp.where(minv == vbest, mini, jnp.int32(_BIGI))
        results.append(jnp.min(cand, axis=1).astype(jnp.int32))
    out_ref[pl.ds(qb, TC_BLK)] = jnp.concatenate(results)


_match_tc = pl.pallas_call(
    _tc_body,
    grid=(TCQ // TC_BLK,),
    in_specs=[
        pl.BlockSpec((TCQ,), lambda i: (0,)),
        pl.BlockSpec((TCQ,), lambda i: (0,)),
        pl.BlockSpec((TCQ,), lambda i: (0,)),
        pl.BlockSpec((NKPAD,), lambda i: (0,)),
        pl.BlockSpec((NKPAD,), lambda i: (0,)),
        pl.BlockSpec((NKPAD,), lambda i: (0,)),
    ],
    out_specs=pl.BlockSpec((TCQ,), lambda i: (0,)),
    out_shape=jax.ShapeDtypeStruct((TCQ,), jnp.int32),
)


def kernel(mhr_vertices, smplx_vertices):
    q = mhr_vertices.astype(jnp.float32)
    s = smplx_vertices.astype(jnp.float32)
    pad = NKPAD - NK
    kx = jnp.pad(s[:, 0], (0, pad), constant_values=1.0e9)
    ky = jnp.pad(s[:, 1], (0, pad), constant_values=1.0e9)
    kz = jnp.pad(s[:, 2], (0, pad), constant_values=1.0e9)
    out_sc = _match_sc(q[:SCQ, 0], q[:SCQ, 1], q[:SCQ, 2], kx, ky, kz)
    out_tc = _match_tc(q[SCQ:, 0], q[SCQ:, 1], q[SCQ:, 2], kx, ky, kz)
    return jnp.concatenate([out_sc, out_tc])


# speedup vs baseline: 1.1192x; 1.0133x over previous
import functools

import jax
import jax.numpy as jnp
from jax import lax
from jax.experimental import pallas as pl
from jax.experimental.pallas import tpu as pltpu
from jax.experimental.pallas import tpu_sc as plsc

NQ = 4096
NK = 10475
LANES = 16
NKPAD = 10496
NCHUNK = NKPAD // LANES
NC = 2
NS = 16
NW = NC * NS

SCQ = 1024
TCQ = NQ - SCQ
QPW = SCQ // NW
TC_BLK = 128

_BIG = 3.0e38
_BIGI = 2**30


@functools.partial(
    pl.kernel,
    out_type=jax.ShapeDtypeStruct((SCQ,), jnp.int32),
    mesh=plsc.VectorSubcoreMesh(core_axis_name="c", subcore_axis_name="s"),
    scratch_types=[
        pltpu.VMEM((NKPAD,), jnp.float32),
        pltpu.VMEM((NKPAD,), jnp.float32),
        pltpu.VMEM((NKPAD,), jnp.float32),
        pltpu.VMEM((QPW,), jnp.float32),
        pltpu.VMEM((QPW,), jnp.float32),
        pltpu.VMEM((QPW,), jnp.float32),
        pltpu.VMEM((QPW,), jnp.int32),
    ],
    compiler_params=pltpu.CompilerParams(needs_layout_passes=False),
)
def _match_sc(qx_hbm, qy_hbm, qz_hbm, kx_hbm, ky_hbm, kz_hbm, out_hbm,
              kxv, kyv, kzv, qxv, qyv, qzv, outv):
    wid = lax.axis_index("s") * NC + lax.axis_index("c")
    base = wid * QPW

    pltpu.sync_copy(kx_hbm, kxv)
    pltpu.sync_copy(ky_hbm, kyv)
    pltpu.sync_copy(kz_hbm, kzv)
    pltpu.sync_copy(qx_hbm.at[pl.ds(base, QPW)], qxv)
    pltpu.sync_copy(qy_hbm.at[pl.ds(base, QPW)], qyv)
    pltpu.sync_copy(qz_hbm.at[pl.ds(base, QPW)], qzv)

    lane_iota = lax.iota(jnp.int32, LANES)

    def group_body(g, carry0):
        acc = jnp.zeros((LANES,), jnp.int32)
        qgx = qxv[pl.ds(g * LANES, LANES)]
        qgy = qyv[pl.ds(g * LANES, LANES)]
        qgz = qzv[pl.ds(g * LANES, LANES)]
        for l in range(LANES):
            qxs = jnp.full((LANES,), qgx[l], jnp.float32)
            qys = jnp.full((LANES,), qgy[l], jnp.float32)
            qzs = jnp.full((LANES,), qgz[l], jnp.float32)

            def key_body(c, carry, qxs=qxs, qys=qys, qzs=qzs):
                minv, mini, idxv = carry
                off = c * LANES
                dx = kxv[pl.ds(off, LANES)] - qxs
                dy = kyv[pl.ds(off, LANES)] - qys
                dz = kzv[pl.ds(off, LANES)] - qzs
                d = dx * dx + dy * dy + dz * dz
                pred = d < minv
                mini = jnp.where(pred, idxv, mini)
                minv = jnp.minimum(d, minv)
                return minv, mini, idxv + LANES

            init = (jnp.full((LANES,), _BIG, jnp.float32),
                    jnp.zeros((LANES,), jnp.int32),
                    lane_iota)
            minv, mini, _ = lax.fori_loop(0, NCHUNK, key_body, init, unroll=16)

            vbest = -plsc.cummax(-minv)[LANES - 1]
            cand = jnp.where(minv == vbest, mini, jnp.int32(_BIGI))
            ibest = -plsc.cummax(-cand)[LANES - 1]
            acc = jnp.where(lane_iota == l, ibest, acc)
        outv[pl.ds(g * LANES, LANES)] = acc
        return carry0

    lax.fori_loop(0, QPW // LANES, group_body, 0)
    pltpu.sync_copy(outv, out_hbm.at[pl.ds(base, QPW)])


TC_KCH = 128


TC_SUB = 64


def _tc_body(qx_ref, qy_ref, qz_ref, kx_ref, ky_ref, kz_ref, out_ref):
    qb = pl.multiple_of(pl.program_id(0) * TC_BLK, TC_BLK)
    qxf = qx_ref[pl.ds(qb, TC_BLK)]
    qyf = qy_ref[pl.ds(qb, TC_BLK)]
    qzf = qz_ref[pl.ds(qb, TC_BLK)]
    lane = lax.broadcasted_iota(jnp.int32, (TC_SUB, TC_KCH), 1)

    results = []
    for h in range(TC_BLK // TC_SUB):
        qx = lax.slice(qxf, (h * TC_SUB,), ((h + 1) * TC_SUB,))[:, None]
        qy = lax.slice(qyf, (h * TC_SUB,), ((h + 1) * TC_SUB,))[:, None]
        qz = lax.slice(qzf, (h * TC_SUB,), ((h + 1) * TC_SUB,))[:, None]

        def body(c, carry, qx=qx, qy=qy, qz=qz):
            minv, minc = carry
            off = c * TC_KCH
            dx = qx - kx_ref[pl.ds(off, TC_KCH)][None, :]
            dy = qy - ky_ref[pl.ds(off, TC_KCH)][None, :]
            dz = qz - kz_ref[pl.ds(off, TC_KCH)][None, :]
            d2 = dx * dx + dy * dy + dz * dz
            pred = d2 < minv
            minc = jnp.where(pred, c, minc)
            minv = jnp.minimum(d2, minv)
            return minv, minc

        init = (jnp.full((TC_SUB, TC_KCH), _BIG, jnp.float32),
                jnp.zeros((TC_SUB, TC_KCH), jnp.int32))
        minv, minc = lax.fori_loop(0, NKPAD // TC_KCH, body, init, unroll=16)
        mini = minc * TC_KCH + lane
        vbest = jnp.min(minv, axis=1, keepdims=True)
        cand = jnp.where(minv == vbest, mini, jnp.int32(_BIGI))
        results.append(jnp.min(cand, axis=1).astype(jnp.int32))
    out_ref[pl.ds(qb, TC_BLK)] = jnp.concatenate(results)


_match_tc = pl.pallas_call(
    _tc_body,
    grid=(TCQ // TC_BLK,),
    in_specs=[
        pl.BlockSpec((TCQ,), lambda i: (0,)),
        pl.BlockSpec((TCQ,), lambda i: (0,)),
        pl.BlockSpec((TCQ,), lambda i: (0,)),
        pl.BlockSpec((NKPAD,), lambda i: (0,)),
        pl.BlockSpec((NKPAD,), lambda i: (0,)),
        pl.BlockSpec((NKPAD,), lambda i: (0,)),
    ],
    out_specs=pl.BlockSpec((TCQ,), lambda i: (0,)),
    out_shape=jax.ShapeDtypeStruct((TCQ,), jnp.int32),
)


def kernel(mhr_vertices, smplx_vertices):
    q = mhr_vertices.astype(jnp.float32)
    s = smplx_vertices.astype(jnp.float32)
    pad = NKPAD - NK
    kx = jnp.pad(s[:, 0], (0, pad), constant_values=1.0e9)
    ky = jnp.pad(s[:, 1], (0, pad), constant_values=1.0e9)
    kz = jnp.pad(s[:, 2], (0, pad), constant_values=1.0e9)
    out_sc = _match_sc(q[:SCQ, 0], q[:SCQ, 1], q[:SCQ, 2], kx, ky, kz)
    out_tc = _match_tc(q[SCQ:, 0], q[SCQ:, 1], q[SCQ:, 2], kx, ky, kz)
    return jnp.concatenate([out_sc, out_tc])


# speedup vs baseline: 1.1215x; 1.0021x over previous
import functools

import jax
import jax.numpy as jnp
from jax import lax
from jax.experimental import pallas as pl
from jax.experimental.pallas import tpu as pltpu
from jax.experimental.pallas import tpu_sc as plsc

NQ = 4096
NK = 10475
LANES = 16
NKPAD = 10496
NCHUNK = NKPAD // LANES
NC = 2
NS = 16
NW = NC * NS

SCQ = 1024
TCQ = NQ - SCQ
QPW = SCQ // NW
TC_BLK = 128

_BIG = 3.0e38
_BIGI = 2**30


@functools.partial(
    pl.kernel,
    out_type=jax.ShapeDtypeStruct((SCQ,), jnp.int32),
    mesh=plsc.VectorSubcoreMesh(core_axis_name="c", subcore_axis_name="s"),
    scratch_types=[
        pltpu.VMEM((NKPAD,), jnp.float32),
        pltpu.VMEM((NKPAD,), jnp.float32),
        pltpu.VMEM((NKPAD,), jnp.float32),
        pltpu.VMEM((QPW,), jnp.float32),
        pltpu.VMEM((QPW,), jnp.float32),
        pltpu.VMEM((QPW,), jnp.float32),
        pltpu.VMEM((QPW,), jnp.int32),
    ],
    compiler_params=pltpu.CompilerParams(needs_layout_passes=False),
)
def _match_sc(qx_hbm, qy_hbm, qz_hbm, kx_hbm, ky_hbm, kz_hbm, out_hbm,
              kxv, kyv, kzv, qxv, qyv, qzv, outv):
    wid = lax.axis_index("s") * NC + lax.axis_index("c")
    base = wid * QPW

    pltpu.sync_copy(kx_hbm, kxv)
    pltpu.sync_copy(ky_hbm, kyv)
    pltpu.sync_copy(kz_hbm, kzv)
    pltpu.sync_copy(qx_hbm.at[pl.ds(base, QPW)], qxv)
    pltpu.sync_copy(qy_hbm.at[pl.ds(base, QPW)], qyv)
    pltpu.sync_copy(qz_hbm.at[pl.ds(base, QPW)], qzv)

    lane_iota = lax.iota(jnp.int32, LANES)

    def group_body(g, carry0):
        acc = jnp.zeros((LANES,), jnp.int32)
        qgx = qxv[pl.ds(g * LANES, LANES)]
        qgy = qyv[pl.ds(g * LANES, LANES)]
        qgz = qzv[pl.ds(g * LANES, LANES)]
        for l in range(LANES):
            qxs = jnp.full((LANES,), qgx[l], jnp.float32)
            qys = jnp.full((LANES,), qgy[l], jnp.float32)
            qzs = jnp.full((LANES,), qgz[l], jnp.float32)

            def key_body(c, carry, qxs=qxs, qys=qys, qzs=qzs):
                minv, mini, idxv = carry
                off = c * LANES
                dx = kxv[pl.ds(off, LANES)] - qxs
                dy = kyv[pl.ds(off, LANES)] - qys
                dz = kzv[pl.ds(off, LANES)] - qzs
                d = dx * dx + dy * dy + dz * dz
                pred = d < minv
                mini = jnp.where(pred, idxv, mini)
                minv = jnp.minimum(d, minv)
                return minv, mini, idxv + LANES

            init = (jnp.full((LANES,), _BIG, jnp.float32),
                    jnp.zeros((LANES,), jnp.int32),
                    lane_iota)
            minv, mini, _ = lax.fori_loop(0, NCHUNK, key_body, init, unroll=16)

            vbest = -plsc.cummax(-minv)[LANES - 1]
            cand = jnp.where(minv == vbest, mini, jnp.int32(_BIGI))
            ibest = -plsc.cummax(-cand)[LANES - 1]
            acc = jnp.where(lane_iota == l, ibest, acc)
        outv[pl.ds(g * LANES, LANES)] = acc
        return carry0

    lax.fori_loop(0, QPW // LANES, group_body, 0)
    pltpu.sync_copy(outv, out_hbm.at[pl.ds(base, QPW)])


TC_KCH = 128


TC_SUB = 64


def _tc_body(qx_ref, qy_ref, qz_ref, kx_ref, ky_ref, kz_ref, out_ref):
    qb = pl.multiple_of(pl.program_id(0) * TC_BLK, TC_BLK)
    qxf = qx_ref[pl.ds(qb, TC_BLK)]
    qyf = qy_ref[pl.ds(qb, TC_BLK)]
    qzf = qz_ref[pl.ds(qb, TC_BLK)]
    lane = lax.broadcasted_iota(jnp.int32, (TC_SUB, TC_KCH), 1)

    partials = []
    for h in range(TC_BLK // TC_SUB):
        qx = lax.slice(qxf, (h * TC_SUB,), ((h + 1) * TC_SUB,))[:, None]
        qy = lax.slice(qyf, (h * TC_SUB,), ((h + 1) * TC_SUB,))[:, None]
        qz = lax.slice(qzf, (h * TC_SUB,), ((h + 1) * TC_SUB,))[:, None]

        def body(c, carry, qx=qx, qy=qy, qz=qz):
            minv, minc = carry
            off = c * TC_KCH
            dx = qx - kx_ref[pl.ds(off, TC_KCH)][None, :]
            dy = qy - ky_ref[pl.ds(off, TC_KCH)][None, :]
            dz = qz - kz_ref[pl.ds(off, TC_KCH)][None, :]
            d2 = dx * dx + dy * dy + dz * dz
            pred = d2 < minv
            minc = jnp.where(pred, c, minc)
            minv = jnp.minimum(d2, minv)
            return minv, minc

        init = (jnp.full((TC_SUB, TC_KCH), _BIG, jnp.float32),
                jnp.zeros((TC_SUB, TC_KCH), jnp.int32))
        partials.append(
            lax.fori_loop(0, NKPAD // TC_KCH, body, init, unroll=16))

    results = []
    for minv, minc in partials:
        mini = minc * TC_KCH + lane
        vbest = jnp.min(minv, axis=1, keepdims=True)
        cand = jnp.where(minv == vbest, mini, jnp.int32(_BIGI))
        results.append(jnp.min(cand, axis=1).astype(jnp.int32))
    out_ref[pl.ds(qb, TC_BLK)] = jnp.concatenate(results)


_match_tc = pl.pallas_call(
    _tc_body,
    grid=(TCQ // TC_BLK,),
    in_specs=[
        pl.BlockSpec((TCQ,), lambda i: (0,)),
        pl.BlockSpec((TCQ,), lambda i: (0,)),
        pl.BlockSpec((TCQ,), lambda i: (0,)),
        pl.BlockSpec((NKPAD,), lambda i: (0,)),
        pl.BlockSpec((NKPAD,), lambda i: (0,)),
        pl.BlockSpec((NKPAD,), lambda i: (0,)),
    ],
    out_specs=pl.BlockSpec((TCQ,), lambda i: (0,)),
    out_shape=jax.ShapeDtypeStruct((TCQ,), jnp.int32),
)


def kernel(mhr_vertices, smplx_vertices):
    q = mhr_vertices.astype(jnp.float32)
    s = smplx_vertices.astype(jnp.float32)
    pad = NKPAD - NK
    kx = jnp.pad(s[:, 0], (0, pad), constant_values=1.0e9)
    ky = jnp.pad(s[:, 1], (0, pad), constant_values=1.0e9)
    kz = jnp.pad(s[:, 2], (0, pad), constant_values=1.0e9)
    out_sc = _match_sc(q[:SCQ, 0], q[:SCQ, 1], q[:SCQ, 2], kx, ky, kz)
    out_tc = _match_tc(q[SCQ:, 0], q[SCQ:, 1], q[SCQ:, 2], kx, ky, kz)
    return jnp.concatenate([out_sc, out_tc])


# speedup vs baseline: 1.1560x; 1.0307x over previous
import functools

import jax
import jax.numpy as jnp
from jax import lax
from jax.experimental import pallas as pl
from jax.experimental.pallas import tpu as pltpu
from jax.experimental.pallas import tpu_sc as plsc

NQ = 4096
NK = 10475
LANES = 16
NKPAD = 10496
NCHUNK = NKPAD // LANES
NC = 2
NS = 16
NW = NC * NS

SCQ = 1024
TCQ = NQ - SCQ
QPW = SCQ // NW
TC_BLK = 128

_BIG = 3.0e38
_BIGI = 2**30


@functools.partial(
    pl.kernel,
    out_type=jax.ShapeDtypeStruct((SCQ,), jnp.int32),
    mesh=plsc.VectorSubcoreMesh(core_axis_name="c", subcore_axis_name="s"),
    scratch_types=[
        pltpu.VMEM((NKPAD,), jnp.float32),
        pltpu.VMEM((NKPAD,), jnp.float32),
        pltpu.VMEM((NKPAD,), jnp.float32),
        pltpu.VMEM((QPW,), jnp.float32),
        pltpu.VMEM((QPW,), jnp.float32),
        pltpu.VMEM((QPW,), jnp.float32),
        pltpu.VMEM((QPW * LANES,), jnp.float32),
        pltpu.VMEM((QPW * LANES,), jnp.float32),
        pltpu.VMEM((QPW * LANES,), jnp.float32),
        pltpu.VMEM((QPW,), jnp.int32),
    ],
    compiler_params=pltpu.CompilerParams(needs_layout_passes=False),
)
def _match_sc(qx_hbm, qy_hbm, qz_hbm, kx_hbm, ky_hbm, kz_hbm, out_hbm,
              kxv, kyv, kzv, qxv, qyv, qzv, qrx, qry, qrz, outv):
    wid = lax.axis_index("s") * NC + lax.axis_index("c")
    base = wid * QPW

    pltpu.sync_copy(kx_hbm, kxv)
    pltpu.sync_copy(ky_hbm, kyv)
    pltpu.sync_copy(kz_hbm, kzv)
    pltpu.sync_copy(qx_hbm.at[pl.ds(base, QPW)], qxv)
    pltpu.sync_copy(qy_hbm.at[pl.ds(base, QPW)], qyv)
    pltpu.sync_copy(qz_hbm.at[pl.ds(base, QPW)], qzv)

    lane_iota = lax.iota(jnp.int32, LANES)

    for g in range(QPW // LANES):
        qgx = qxv[pl.ds(g * LANES, LANES)]
        qgy = qyv[pl.ds(g * LANES, LANES)]
        qgz = qzv[pl.ds(g * LANES, LANES)]
        for l in range(LANES):
            off = (g * LANES + l) * LANES
            qrx[pl.ds(off, LANES)] = jnp.full((LANES,), qgx[l], jnp.float32)
            qry[pl.ds(off, LANES)] = jnp.full((LANES,), qgy[l], jnp.float32)
            qrz[pl.ds(off, LANES)] = jnp.full((LANES,), qgz[l], jnp.float32)

    def query_body(q, carry0):
        qxs = qrx[pl.ds(q * LANES, LANES)]
        qys = qry[pl.ds(q * LANES, LANES)]
        qzs = qrz[pl.ds(q * LANES, LANES)]

        def key_body(c, carry):
            minv, mini, idxv = carry
            off = c * LANES
            dx = kxv[pl.ds(off, LANES)] - qxs
            dy = kyv[pl.ds(off, LANES)] - qys
            dz = kzv[pl.ds(off, LANES)] - qzs
            d = dx * dx + dy * dy + dz * dz
            pred = d < minv
            mini = jnp.where(pred, idxv, mini)
            minv = jnp.minimum(d, minv)
            return minv, mini, idxv + LANES

        init = (jnp.full((LANES,), _BIG, jnp.float32),
                jnp.zeros((LANES,), jnp.int32),
                lane_iota)
        minv, mini, _ = lax.fori_loop(0, NCHUNK, key_body, init, unroll=16)

        vbest = -plsc.cummax(-minv)[LANES - 1]
        cand = jnp.where(minv == vbest, mini, jnp.int32(_BIGI))
        ibest_vec = -plsc.cummax(-cand)
        plsc.store_scatter(outv, [jnp.full((LANES,), q, jnp.int32)],
                           ibest_vec, mask=lane_iota == LANES - 1)
        return carry0

    lax.fori_loop(0, QPW, query_body, 0)
    pltpu.sync_copy(outv, out_hbm.at[pl.ds(base, QPW)])


TC_KCH = 128


TC_SUB = 64


def _tc_body(qx_ref, qy_ref, qz_ref, kx_ref, ky_ref, kz_ref, out_ref):
    qb = pl.multiple_of(pl.program_id(0) * TC_BLK, TC_BLK)
    qxf = qx_ref[pl.ds(qb, TC_BLK)]
    qyf = qy_ref[pl.ds(qb, TC_BLK)]
    qzf = qz_ref[pl.ds(qb, TC_BLK)]
    lane = lax.broadcasted_iota(jnp.int32, (TC_SUB, TC_KCH), 1)

    partials = []
    for h in range(TC_BLK // TC_SUB):
        qx = lax.slice(qxf, (h * TC_SUB,), ((h + 1) * TC_SUB,))[:, None]
        qy = lax.slice(qyf, (h * TC_SUB,), ((h + 1) * TC_SUB,))[:, None]
        qz = lax.slice(qzf, (h * TC_SUB,), ((h + 1) * TC_SUB,))[:, None]

        def body(c, carry, qx=qx, qy=qy, qz=qz):
            minv, minc = carry
            off = c * TC_KCH
            dx = qx - kx_ref[pl.ds(off, TC_KCH)][None, :]
            dy = qy - ky_ref[pl.ds(off, TC_KCH)][None, :]
            dz = qz - kz_ref[pl.ds(off, TC_KCH)][None, :]
            d2 = dx * dx + dy * dy + dz * dz
            pred = d2 < minv
            minc = jnp.where(pred, c, minc)
            minv = jnp.minimum(d2, minv)
            return minv, minc

        init = (jnp.full((TC_SUB, TC_KCH), _BIG, jnp.float32),
                jnp.zeros((TC_SUB, TC_KCH), jnp.int32))
        partials.append(
            lax.fori_loop(0, NKPAD // TC_KCH, body, init, unroll=16))

    results = []
    for minv, minc in partials:
        mini = minc * TC_KCH + lane
        vbest = jnp.min(minv, axis=1, keepdims=True)
        cand = jnp.where(minv == vbest, mini, jnp.int32(_BIGI))
        results.append(jnp.min(cand, axis=1).astype(jnp.int32))
    out_ref[pl.ds(qb, TC_BLK)] = jnp.concatenate(results)


_match_tc = pl.pallas_call(
    _tc_body,
    grid=(TCQ // TC_BLK,),
    in_specs=[
        pl.BlockSpec((TCQ,), lambda i: (0,)),
        pl.BlockSpec((TCQ,), lambda i: (0,)),
        pl.BlockSpec((TCQ,), lambda i: (0,)),
        pl.BlockSpec((NKPAD,), lambda i: (0,)),
        pl.BlockSpec((NKPAD,), lambda i: (0,)),
        pl.BlockSpec((NKPAD,), lambda i: (0,)),
    ],
    out_specs=pl.BlockSpec((TCQ,), lambda i: (0,)),
    out_shape=jax.ShapeDtypeStruct((TCQ,), jnp.int32),
)


def kernel(mhr_vertices, smplx_vertices):
    q = mhr_vertices.astype(jnp.float32)
    s = smplx_vertices.astype(jnp.float32)
    pad = NKPAD - NK
    kx = jnp.pad(s[:, 0], (0, pad), constant_values=1.0e9)
    ky = jnp.pad(s[:, 1], (0, pad), constant_values=1.0e9)
    kz = jnp.pad(s[:, 2], (0, pad), constant_values=1.0e9)
    out_sc = _match_sc(q[:SCQ, 0], q[:SCQ, 1], q[:SCQ, 2], kx, ky, kz)
    out_tc = _match_tc(q[SCQ:, 0], q[SCQ:, 1], q[SCQ:, 2], kx, ky, kz)
    return jnp.concatenate([out_sc, out_tc])


# speedup vs baseline: 1.1822x; 1.0227x over previous
import functools

import jax
import jax.numpy as jnp
from jax import lax
from jax.experimental import pallas as pl
from jax.experimental.pallas import tpu as pltpu
from jax.experimental.pallas import tpu_sc as plsc

NQ = 4096
NK = 10475
LANES = 16
NKPAD = 10496
NCHUNK = NKPAD // LANES
NC = 2
NS = 16
NW = NC * NS

SCQ = 1024
TCQ = NQ - SCQ
QPW = SCQ // NW
TC_BLK = 128

_BIG = 3.0e38
_BIGI = 2**30


@functools.partial(
    pl.kernel,
    out_type=jax.ShapeDtypeStruct((SCQ,), jnp.int32),
    mesh=plsc.VectorSubcoreMesh(core_axis_name="c", subcore_axis_name="s"),
    scratch_types=[
        pltpu.VMEM((NKPAD,), jnp.float32),
        pltpu.VMEM((NKPAD,), jnp.float32),
        pltpu.VMEM((NKPAD,), jnp.float32),
        pltpu.VMEM((QPW,), jnp.float32),
        pltpu.VMEM((QPW,), jnp.float32),
        pltpu.VMEM((QPW,), jnp.float32),
        pltpu.VMEM((QPW * LANES,), jnp.float32),
        pltpu.VMEM((QPW * LANES,), jnp.float32),
        pltpu.VMEM((QPW * LANES,), jnp.float32),
        pltpu.VMEM((QPW,), jnp.int32),
    ],
    compiler_params=pltpu.CompilerParams(needs_layout_passes=False),
)
def _match_sc(qx_hbm, qy_hbm, qz_hbm, kx_hbm, ky_hbm, kz_hbm, out_hbm,
              kxv, kyv, kzv, qxv, qyv, qzv, qrx, qry, qrz, outv):
    wid = lax.axis_index("s") * NC + lax.axis_index("c")
    base = wid * QPW

    pltpu.sync_copy(kx_hbm, kxv)
    pltpu.sync_copy(ky_hbm, kyv)
    pltpu.sync_copy(kz_hbm, kzv)
    pltpu.sync_copy(qx_hbm.at[pl.ds(base, QPW)], qxv)
    pltpu.sync_copy(qy_hbm.at[pl.ds(base, QPW)], qyv)
    pltpu.sync_copy(qz_hbm.at[pl.ds(base, QPW)], qzv)

    lane_iota = lax.iota(jnp.int32, LANES)

    for g in range(QPW // LANES):
        qgx = qxv[pl.ds(g * LANES, LANES)]
        qgy = qyv[pl.ds(g * LANES, LANES)]
        qgz = qzv[pl.ds(g * LANES, LANES)]
        for l in range(LANES):
            off = (g * LANES + l) * LANES
            qrx[pl.ds(off, LANES)] = jnp.full((LANES,), qgx[l], jnp.float32)
            qry[pl.ds(off, LANES)] = jnp.full((LANES,), qgy[l], jnp.float32)
            qrz[pl.ds(off, LANES)] = jnp.full((LANES,), qgz[l], jnp.float32)

    def query_body(q, carry0):
        qxs = qrx[pl.ds(q * LANES, LANES)]
        qys = qry[pl.ds(q * LANES, LANES)]
        qzs = qrz[pl.ds(q * LANES, LANES)]

        def key_body(c, carry):
            minv, mini, idxv = carry
            off = c * LANES
            dx = kxv[pl.ds(off, LANES)] - qxs
            dy = kyv[pl.ds(off, LANES)] - qys
            dz = kzv[pl.ds(off, LANES)] - qzs
            d = dx * dx + dy * dy + dz * dz
            pred = d < minv
            mini = jnp.where(pred, idxv, mini)
            minv = jnp.minimum(d, minv)
            return minv, mini, idxv + LANES

        init = (jnp.full((LANES,), _BIG, jnp.float32),
                jnp.zeros((LANES,), jnp.int32),
                lane_iota)
        minv, mini, _ = lax.fori_loop(0, NCHUNK, key_body, init, unroll=16)

        vbest = -plsc.cummax(-minv)[LANES - 1]
        cand = jnp.where(minv == vbest, mini, jnp.int32(_BIGI))
        ibest_vec = -plsc.cummax(-cand)
        plsc.store_scatter(outv, [jnp.full((LANES,), q, jnp.int32)],
                           ibest_vec, mask=lane_iota == LANES - 1)
        return carry0

    lax.fori_loop(0, QPW, query_body, 0)
    pltpu.sync_copy(outv, out_hbm.at[pl.ds(base, QPW)])


TC_KCH = 128


TC_SUB = 64


def _tc_body(qx_ref, qy_ref, qz_ref, kx_ref, ky_ref, kz_ref, out_ref):
    qb = pl.multiple_of(pl.program_id(0) * TC_BLK, TC_BLK)
    qxf = qx_ref[pl.ds(qb, TC_BLK)]
    qyf = qy_ref[pl.ds(qb, TC_BLK)]
    qzf = qz_ref[pl.ds(qb, TC_BLK)]
    lane = lax.broadcasted_iota(jnp.int32, (TC_SUB, TC_KCH), 1)

    partials = []
    for h in range(TC_BLK // TC_SUB):
        qx = lax.slice(qxf, (h * TC_SUB,), ((h + 1) * TC_SUB,))[:, None]
        qy = lax.slice(qyf, (h * TC_SUB,), ((h + 1) * TC_SUB,))[:, None]
        qz = lax.slice(qzf, (h * TC_SUB,), ((h + 1) * TC_SUB,))[:, None]

        def body(c, carry, qx=qx, qy=qy, qz=qz):
            minv, minc = carry
            off = c * TC_KCH
            dx = qx - kx_ref[pl.ds(off, TC_KCH)][None, :]
            dy = qy - ky_ref[pl.ds(off, TC_KCH)][None, :]
            dz = qz - kz_ref[pl.ds(off, TC_KCH)][None, :]
            d2 = dx * dx + dy * dy + dz * dz
            pred = d2 < minv
            minc = jnp.where(pred, c, minc)
            minv = jnp.minimum(d2, minv)
            return minv, minc

        init = (jnp.full((TC_SUB, TC_KCH), _BIG, jnp.float32),
                jnp.zeros((TC_SUB, TC_KCH), jnp.int32))
        partials.append(
            lax.fori_loop(0, NKPAD // TC_KCH, body, init, unroll=16))

    results = []
    for minv, minc in partials:
        mini = minc * TC_KCH + lane
        vbest = jnp.min(minv, axis=1, keepdims=True)
        cand = jnp.where(minv == vbest, mini, jnp.int32(_BIGI))
        results.append(jnp.min(cand, axis=1).astype(jnp.int32))
    out_ref[pl.ds(qb, TC_BLK)] = jnp.concatenate(results)


_match_tc = pl.pallas_call(
    _tc_body,
    grid=(TCQ // TC_BLK,),
    in_specs=[
        pl.BlockSpec((TCQ,), lambda i: (0,)),
        pl.BlockSpec((TCQ,), lambda i: (0,)),
        pl.BlockSpec((TCQ,), lambda i: (0,)),
        pl.BlockSpec((NKPAD,), lambda i: (0,)),
        pl.BlockSpec((NKPAD,), lambda i: (0,)),
        pl.BlockSpec((NKPAD,), lambda i: (0,)),
    ],
    out_specs=pl.BlockSpec((TCQ,), lambda i: (0,)),
    out_shape=jax.ShapeDtypeStruct((TCQ,), jnp.int32),
)


def kernel(mhr_vertices, smplx_vertices):
    q = mhr_vertices.astype(jnp.float32)
    s = smplx_vertices.astype(jnp.float32)
    sp = jnp.pad(s, ((0, NKPAD - NK), (0, 0)), constant_values=1.0e9)
    kx, ky, kz = sp[:, 0], sp[:, 1], sp[:, 2]
    out_sc = _match_sc(q[:SCQ, 0], q[:SCQ, 1], q[:SCQ, 2], kx, ky, kz)
    out_tc = _match_tc(q[SCQ:, 0], q[SCQ:, 1], q[SCQ:, 2], kx, ky, kz)
    return jnp.concatenate([out_sc, out_tc])


# speedup vs baseline: 1.1848x; 1.0023x over previous
import functools

import jax
import jax.numpy as jnp
from jax import lax
from jax.experimental import pallas as pl
from jax.experimental.pallas import tpu as pltpu
from jax.experimental.pallas import tpu_sc as plsc

NQ = 4096
NK = 10475
LANES = 16
NKPAD = 10496
NCHUNK = NKPAD // LANES
NC = 2
NS = 16
NW = NC * NS

SCQ = 1024
TCQ = NQ - SCQ
QPW = SCQ // NW
TC_BLK = 128

_BIG = 3.0e38
_BIGI = 2**30


@functools.partial(
    pl.kernel,
    out_type=jax.ShapeDtypeStruct((SCQ,), jnp.int32),
    mesh=plsc.VectorSubcoreMesh(core_axis_name="c", subcore_axis_name="s"),
    scratch_types=[
        pltpu.VMEM((NKPAD,), jnp.float32),
        pltpu.VMEM((NKPAD,), jnp.float32),
        pltpu.VMEM((NKPAD,), jnp.float32),
        pltpu.VMEM((QPW,), jnp.float32),
        pltpu.VMEM((QPW,), jnp.float32),
        pltpu.VMEM((QPW,), jnp.float32),
        pltpu.VMEM((QPW * LANES,), jnp.float32),
        pltpu.VMEM((QPW * LANES,), jnp.float32),
        pltpu.VMEM((QPW * LANES,), jnp.float32),
        pltpu.VMEM((QPW,), jnp.int32),
    ],
    compiler_params=pltpu.CompilerParams(needs_layout_passes=False),
)
def _match_sc(qx_hbm, qy_hbm, qz_hbm, kx_hbm, ky_hbm, kz_hbm, out_hbm,
              kxv, kyv, kzv, qxv, qyv, qzv, qrx, qry, qrz, outv):
    wid = lax.axis_index("s") * NC + lax.axis_index("c")
    base = wid * QPW

    pltpu.sync_copy(kx_hbm, kxv)
    pltpu.sync_copy(ky_hbm, kyv)
    pltpu.sync_copy(kz_hbm, kzv)
    pltpu.sync_copy(qx_hbm.at[pl.ds(base, QPW)], qxv)
    pltpu.sync_copy(qy_hbm.at[pl.ds(base, QPW)], qyv)
    pltpu.sync_copy(qz_hbm.at[pl.ds(base, QPW)], qzv)

    lane_iota = lax.iota(jnp.int32, LANES)

    for g in range(QPW // LANES):
        qgx = qxv[pl.ds(g * LANES, LANES)]
        qgy = qyv[pl.ds(g * LANES, LANES)]
        qgz = qzv[pl.ds(g * LANES, LANES)]
        for l in range(LANES):
            off = (g * LANES + l) * LANES
            qrx[pl.ds(off, LANES)] = jnp.full((LANES,), qgx[l], jnp.float32)
            qry[pl.ds(off, LANES)] = jnp.full((LANES,), qgy[l], jnp.float32)
            qrz[pl.ds(off, LANES)] = jnp.full((LANES,), qgz[l], jnp.float32)

    def query_body(q, carry0):
        qxs = qrx[pl.ds(q * LANES, LANES)]
        qys = qry[pl.ds(q * LANES, LANES)]
        qzs = qrz[pl.ds(q * LANES, LANES)]

        def key_body(c, carry):
            minv, mini, idxv = carry
            off = c * LANES
            dx = kxv[pl.ds(off, LANES)] - qxs
            dy = kyv[pl.ds(off, LANES)] - qys
            dz = kzv[pl.ds(off, LANES)] - qzs
            d = dx * dx + dy * dy + dz * dz
            pred = d < minv
            mini = jnp.where(pred, idxv, mini)
            minv = jnp.minimum(d, minv)
            return minv, mini, idxv + LANES

        init = (jnp.full((LANES,), _BIG, jnp.float32),
                jnp.zeros((LANES,), jnp.int32),
                lane_iota)
        minv, mini, _ = lax.fori_loop(0, NCHUNK, key_body, init, unroll=8)

        vbest = -plsc.cummax(-minv)[LANES - 1]
        cand = jnp.where(minv == vbest, mini, jnp.int32(_BIGI))
        ibest_vec = -plsc.cummax(-cand)
        plsc.store_scatter(outv, [jnp.full((LANES,), q, jnp.int32)],
                           ibest_vec, mask=lane_iota == LANES - 1)
        return carry0

    lax.fori_loop(0, QPW, query_body, 0)
    pltpu.sync_copy(outv, out_hbm.at[pl.ds(base, QPW)])


TC_KCH = 128


TC_SUB = 64


def _tc_body(qx_ref, qy_ref, qz_ref, kx_ref, ky_ref, kz_ref, out_ref):
    qb = pl.multiple_of(pl.program_id(0) * TC_BLK, TC_BLK)
    qxf = qx_ref[pl.ds(qb, TC_BLK)]
    qyf = qy_ref[pl.ds(qb, TC_BLK)]
    qzf = qz_ref[pl.ds(qb, TC_BLK)]
    lane = lax.broadcasted_iota(jnp.int32, (TC_SUB, TC_KCH), 1)

    partials = []
    for h in range(TC_BLK // TC_SUB):
        qx = lax.slice(qxf, (h * TC_SUB,), ((h + 1) * TC_SUB,))[:, None]
        qy = lax.slice(qyf, (h * TC_SUB,), ((h + 1) * TC_SUB,))[:, None]
        qz = lax.slice(qzf, (h * TC_SUB,), ((h + 1) * TC_SUB,))[:, None]

        def body(c, carry, qx=qx, qy=qy, qz=qz):
            minv, minc = carry
            off = c * TC_KCH
            dx = qx - kx_ref[pl.ds(off, TC_KCH)][None, :]
            dy = qy - ky_ref[pl.ds(off, TC_KCH)][None, :]
            dz = qz - kz_ref[pl.ds(off, TC_KCH)][None, :]
            d2 = dx * dx + dy * dy + dz * dz
            pred = d2 < minv
            minc = jnp.where(pred, c, minc)
            minv = jnp.minimum(d2, minv)
            return minv, minc

        init = (jnp.full((TC_SUB, TC_KCH), _BIG, jnp.float32),
                jnp.zeros((TC_SUB, TC_KCH), jnp.int32))
        partials.append(
            lax.fori_loop(0, NKPAD // TC_KCH, body, init, unroll=16))

    results = []
    for minv, minc in partials:
        mini = minc * TC_KCH + lane
        vbest = jnp.min(minv, axis=1, keepdims=True)
        cand = jnp.where(minv == vbest, mini, jnp.int32(_BIGI))
        results.append(jnp.min(cand, axis=1).astype(jnp.int32))
    out_ref[pl.ds(qb, TC_BLK)] = jnp.concatenate(results)


_match_tc = pl.pallas_call(
    _tc_body,
    grid=(TCQ // TC_BLK,),
    in_specs=[
        pl.BlockSpec((TCQ,), lambda i: (0,)),
        pl.BlockSpec((TCQ,), lambda i: (0,)),
        pl.BlockSpec((TCQ,), lambda i: (0,)),
        pl.BlockSpec((NKPAD,), lambda i: (0,)),
        pl.BlockSpec((NKPAD,), lambda i: (0,)),
        pl.BlockSpec((NKPAD,), lambda i: (0,)),
    ],
    out_specs=pl.BlockSpec((TCQ,), lambda i: (0,)),
    out_shape=jax.ShapeDtypeStruct((TCQ,), jnp.int32),
)


def kernel(mhr_vertices, smplx_vertices):
    q = mhr_vertices.astype(jnp.float32)
    s = smplx_vertices.astype(jnp.float32)
    sp = jnp.pad(s, ((0, NKPAD - NK), (0, 0)), constant_values=1.0e9)
    kx, ky, kz = sp[:, 0], sp[:, 1], sp[:, 2]
    out_sc = _match_sc(q[:SCQ, 0], q[:SCQ, 1], q[:SCQ, 2], kx, ky, kz)
    out_tc = _match_tc(q[SCQ:, 0], q[SCQ:, 1], q[SCQ:, 2], kx, ky, kz)
    return jnp.concatenate([out_sc, out_tc])
